# baseline (speedup 1.0000x reference)
import functools

import jax
import jax.numpy as jnp
from jax import lax
from jax.experimental import pallas as pl
from jax.experimental.pallas import tpu as pltpu
from jax.experimental.pallas import tpu_sc as plsc

LANES = 16
IDX_CHUNK = 128
NF = 16


def _sc_dims():
    try:
        info = plsc.get_sparse_core_info()
        return info.num_cores, info.num_subcores
    except Exception:
        return 2, 16


def _make_body(nc, nchunk, bpw):
    def body(users_hbm, items_hbm, uft_hbm, ift_hbm, out_hbm,
             idx_u, idx_v, u_cols, v_cols, out_v, sem):
        wid = lax.axis_index("s") * nc + lax.axis_index("c")
        base = wid * bpw

        cp_u = pltpu.async_copy(users_hbm.at[wid], idx_u, sem)
        cp_v = pltpu.async_copy(items_hbm.at[wid], idx_v, sem)
        cp_u.wait()
        cp_v.wait()

        for j in range(nchunk):
            handles = []
            sl = pl.ds(j * IDX_CHUNK, IDX_CHUNK)
            for k in range(NF):
                handles.append(pltpu.async_copy(
                    uft_hbm.at[k].at[idx_u.at[j]], u_cols.at[k].at[sl], sem))
                handles.append(pltpu.async_copy(
                    ift_hbm.at[k].at[idx_v.at[j]], v_cols.at[k].at[sl], sem))
            for h in handles:
                h.wait()

        def blk(b, _):
            sl = pl.ds(b * LANES, LANES)
            acc = u_cols[0, sl] * v_cols[0, sl]
            for k in range(1, NF):
                acc = acc + u_cols[k, sl] * v_cols[k, sl]
            out_v[sl] = acc
            return 0

        lax.fori_loop(0, bpw // LANES, blk, 0)
        pltpu.sync_copy(out_v, out_hbm.at[pl.ds(base, bpw)])

    return body


@jax.jit
def kernel(x, user_factors, item_factors):
    nc, ns = _sc_dims()
    nw = nc * ns
    batch = x.shape[0]
    assert batch % (nw * IDX_CHUNK) == 0
    bpw = batch // nw
    nchunk = bpw // IDX_CHUNK

    users = x[:, 0].astype(jnp.int32).reshape(nw, nchunk, IDX_CHUNK)
    items = x[:, 1].astype(jnp.int32).reshape(nw, nchunk, IDX_CHUNK)

    mesh = plsc.VectorSubcoreMesh(core_axis_name="c", subcore_axis_name="s")
    fn = pl.kernel(
        _make_body(nc, nchunk, bpw),
        out_type=jax.ShapeDtypeStruct((batch,), jnp.float32),
        mesh=mesh,
        scratch_types=[
            pltpu.VMEM((nchunk, IDX_CHUNK), jnp.int32),
            pltpu.VMEM((nchunk, IDX_CHUNK), jnp.int32),
            pltpu.VMEM((NF, bpw), jnp.float32),
            pltpu.VMEM((NF, bpw), jnp.float32),
            pltpu.VMEM((bpw,), jnp.float32),
            pltpu.SemaphoreType.DMA,
        ],
        compiler_params=pltpu.CompilerParams(use_tc_tiling_on_sc=False),
    )
    return fn(users, items, user_factors.T, item_factors.T)


# speedup vs baseline: 17.4463x; 17.4463x over previous
import functools

import jax
import jax.numpy as jnp
from jax import lax
from jax.experimental import pallas as pl
from jax.experimental.pallas import tpu as pltpu
from jax.experimental.pallas import tpu_sc as plsc

LANES = 16
NF = 16
NBUF = 4
SPS = 4
GRP = NBUF * SPS


def _sc_dims():
    try:
        info = plsc.get_sparse_core_info()
        return info.num_cores, info.num_subcores
    except Exception:
        return 2, 16


def _make_body(nc, bpw, vocab):
    ngrp = bpw // GRP

    def body(users_hbm, items_hbm, uft_hbm, ift_hbm, out_hbm,
             idx_u, idx_v, u_bufs, v_bufs, out_v, *sems):
        wid = lax.axis_index("s") * nc + lax.axis_index("c")
        base = wid * bpw

        cp_u = pltpu.async_copy(users_hbm.at[wid], idx_u, sems[0])
        cp_v = pltpu.async_copy(items_hbm.at[wid], idx_v, sems[1])
        cp_u.wait()
        cp_v.wait()

        lane = lax.iota(jnp.int32, LANES)
        perms = [lane ^ d for d in (1, 2, 4, 8)]

        def block_base(i):
            return pl.multiple_of((i // 128) * 128, 128)

        def fire_batch(s, iv_u, iv_v, t0):
            for t in range(SPS):
                iu = iv_u[t0 + t]
                ivv = iv_v[t0 + t]
                pltpu.async_copy(
                    uft_hbm.at[:, pl.ds(block_base(iu), 128)],
                    u_bufs.at[s * SPS + t], sems[s])
                pltpu.async_copy(
                    ift_hbm.at[:, pl.ds(block_base(ivv), 128)],
                    v_bufs.at[s * SPS + t], sems[s])

        def drain_batch(s):
            for t in range(SPS):
                pltpu.make_async_copy(
                    uft_hbm.at[:, pl.ds(0, 128)], u_bufs.at[s * SPS + t],
                    sems[s]).wait()
                pltpu.make_async_copy(
                    ift_hbm.at[:, pl.ds(0, 128)], v_bufs.at[s * SPS + t],
                    sems[s]).wait()

        iv_u0 = idx_u[pl.ds(0, GRP)]
        iv_v0 = idx_v[pl.ds(0, GRP)]
        for s in range(NBUF):
            fire_batch(s, iv_u0, iv_v0, s * SPS)

        def grp(g, _):
            goff = g * GRP
            iv_u = idx_u[pl.ds(goff, GRP)]
            iv_v = idx_v[pl.ds(goff, GRP)]
            acc = jnp.zeros((LANES,), jnp.float32)
            for s in range(NBUF):
                drain_batch(s)
                for t in range(SPS):
                    j = s * SPS + t
                    iu = iv_u[j]
                    ivv = iv_v[j]
                    lu = jnp.broadcast_to(iu - block_base(iu), (LANES,))
                    lv = jnp.broadcast_to(ivv - block_base(ivv), (LANES,))
                    gu = plsc.load_gather(u_bufs.at[s * SPS + t], [lane, lu])
                    gv = plsc.load_gather(v_bufs.at[s * SPS + t], [lane, lv])
                    p = gu * gv
                    for perm in perms:
                        p = p + p.at[perm].get(mode="promise_in_bounds")
                    acc = jnp.where(lane == j, p, acc)

            @pl.when(g + 1 < ngrp)
            def _refire():
                nof = (g + 1) * GRP
                nu = idx_u[pl.ds(nof, GRP)]
                nv = idx_v[pl.ds(nof, GRP)]
                for s in range(NBUF):
                    fire_batch(s, nu, nv, s * SPS)

            out_v[pl.ds(goff, GRP)] = acc
            return 0

        lax.fori_loop(0, ngrp, grp, 0)
        pltpu.sync_copy(out_v, out_hbm.at[pl.ds(base, bpw)])

    return body


@jax.jit
def kernel(x, user_factors, item_factors):
    nc, ns = _sc_dims()
    nw = nc * ns
    batch = x.shape[0]
    vocab = user_factors.shape[0]
    assert batch % (nw * GRP) == 0
    bpw = batch // nw

    users = x[:, 0].astype(jnp.int32).reshape(nw, bpw)
    items = x[:, 1].astype(jnp.int32).reshape(nw, bpw)

    mesh = plsc.VectorSubcoreMesh(core_axis_name="c", subcore_axis_name="s")
    fn = pl.kernel(
        _make_body(nc, bpw, vocab),
        out_type=jax.ShapeDtypeStruct((batch,), jnp.float32),
        mesh=mesh,
        scratch_types=[
            pltpu.VMEM((bpw,), jnp.int32),
            pltpu.VMEM((bpw,), jnp.int32),
            pltpu.VMEM((NBUF * SPS, NF, 128), jnp.float32),
            pltpu.VMEM((NBUF * SPS, NF, 128), jnp.float32),
            pltpu.VMEM((bpw,), jnp.float32),
        ] + [pltpu.SemaphoreType.DMA] * NBUF,
        compiler_params=pltpu.CompilerParams(
            disable_bounds_checks=True, needs_layout_passes=False),
    )
    return fn(users, items, user_factors.T, item_factors.T)


# speedup vs baseline: 20.9037x; 1.1982x over previous
import functools

import jax
import jax.numpy as jnp
from jax import lax
from jax.experimental import pallas as pl
from jax.experimental.pallas import tpu as pltpu
from jax.experimental.pallas import tpu_sc as plsc

LANES = 16
NF = 16
NBUF = 4
SPS = 4
GRP = NBUF * SPS


def _sc_dims():
    try:
        info = plsc.get_sparse_core_info()
        return info.num_cores, info.num_subcores
    except Exception:
        return 2, 16


def _make_body(nc, bpw, vocab):
    ngrp = bpw // GRP

    def body(users_hbm, items_hbm, uft_hbm, ift_hbm, out_hbm,
             idx_u, idx_v, u_bufs, v_bufs, out_v, *sems):
        wid = lax.axis_index("s") * nc + lax.axis_index("c")
        base = wid * bpw

        cp_u = pltpu.async_copy(users_hbm.at[wid], idx_u, sems[0])
        cp_v = pltpu.async_copy(items_hbm.at[wid], idx_v, sems[1])
        cp_u.wait()
        cp_v.wait()

        lane = lax.iota(jnp.int32, LANES)
        perms = [lane ^ d for d in (1, 2, 4, 8)]

        def block_base(i):
            return pl.multiple_of((i // 128) * 128, 128)

        def fire_batch(s, iv_u, iv_v, t0):
            for t in range(SPS):
                iu = iv_u[t0 + t]
                ivv = iv_v[t0 + t]
                pltpu.async_copy(
                    uft_hbm.at[:, pl.ds(block_base(iu), 128)],
                    u_bufs.at[s * SPS + t], sems[s])
                pltpu.async_copy(
                    ift_hbm.at[:, pl.ds(block_base(ivv), 128)],
                    v_bufs.at[s * SPS + t], sems[s])

        def drain_batch(s):
            for t in range(SPS):
                pltpu.make_async_copy(
                    uft_hbm.at[:, pl.ds(0, 128)], u_bufs.at[s * SPS + t],
                    sems[s]).wait()
                pltpu.make_async_copy(
                    ift_hbm.at[:, pl.ds(0, 128)], v_bufs.at[s * SPS + t],
                    sems[s]).wait()

        iv_u0 = idx_u[pl.ds(0, GRP)]
        iv_v0 = idx_v[pl.ds(0, GRP)]
        for s in range(NBUF):
            fire_batch(s, iv_u0, iv_v0, s * SPS)

        def grp(g, _):
            goff = g * GRP
            iv_u = idx_u[pl.ds(goff, GRP)]
            iv_v = idx_v[pl.ds(goff, GRP)]
            nof = jnp.minimum(g + 1, ngrp - 1) * GRP
            nu = idx_u[pl.ds(nof, GRP)]
            nv = idx_v[pl.ds(nof, GRP)]
            acc = jnp.zeros((LANES,), jnp.float32)
            for s in range(NBUF):
                drain_batch(s)
                for t in range(SPS):
                    j = s * SPS + t
                    iu = iv_u[j]
                    ivv = iv_v[j]
                    lu = jnp.broadcast_to(iu - block_base(iu), (LANES,))
                    lv = jnp.broadcast_to(ivv - block_base(ivv), (LANES,))
                    gu = plsc.load_gather(u_bufs.at[s * SPS + t], [lane, lu])
                    gv = plsc.load_gather(v_bufs.at[s * SPS + t], [lane, lv])
                    p = gu * gv
                    for perm in perms:
                        p = p + p.at[perm].get(mode="promise_in_bounds")
                    acc = jnp.where(lane == j, p, acc)

                @pl.when(g + 1 < ngrp)
                def _refire():
                    fire_batch(s, nu, nv, s * SPS)

            out_v[pl.ds(goff, GRP)] = acc
            return 0

        lax.fori_loop(0, ngrp, grp, 0)
        pltpu.sync_copy(out_v, out_hbm.at[pl.ds(base, bpw)])

    return body


@jax.jit
def kernel(x, user_factors, item_factors):
    nc, ns = _sc_dims()
    nw = nc * ns
    batch = x.shape[0]
    vocab = user_factors.shape[0]
    assert batch % (nw * GRP) == 0
    bpw = batch // nw

    users = x[:, 0].astype(jnp.int32).reshape(nw, bpw)
    items = x[:, 1].astype(jnp.int32).reshape(nw, bpw)

    mesh = plsc.VectorSubcoreMesh(core_axis_name="c", subcore_axis_name="s")
    fn = pl.kernel(
        _make_body(nc, bpw, vocab),
        out_type=jax.ShapeDtypeStruct((batch,), jnp.float32),
        mesh=mesh,
        scratch_types=[
            pltpu.VMEM((bpw,), jnp.int32),
            pltpu.VMEM((bpw,), jnp.int32),
            pltpu.VMEM((NBUF * SPS, NF, 128), jnp.float32),
            pltpu.VMEM((NBUF * SPS, NF, 128), jnp.float32),
            pltpu.VMEM((bpw,), jnp.float32),
        ] + [pltpu.SemaphoreType.DMA] * NBUF,
        compiler_params=pltpu.CompilerParams(
            disable_bounds_checks=True, needs_layout_passes=False),
    )
    return fn(users, items, user_factors.T, item_factors.T)


# speedup vs baseline: 22.2164x; 1.0628x over previous
import functools

import jax
import jax.numpy as jnp
from jax import lax
from jax.experimental import pallas as pl
from jax.experimental.pallas import tpu as pltpu
from jax.experimental.pallas import tpu_sc as plsc

LANES = 16
NF = 16
NBUF = 8
SPS = 2
GRP = NBUF * SPS


def _sc_dims():
    try:
        info = plsc.get_sparse_core_info()
        return info.num_cores, info.num_subcores
    except Exception:
        return 2, 16


def _make_body(nc, bpw, vocab):
    ngrp = bpw // GRP

    def body(users_hbm, items_hbm, uft_hbm, ift_hbm, out_hbm,
             idx_u, idx_v, u_bufs, v_bufs, out_v, *sems):
        wid = lax.axis_index("s") * nc + lax.axis_index("c")
        base = wid * bpw

        cp_u = pltpu.async_copy(users_hbm.at[wid], idx_u, sems[0])
        cp_v = pltpu.async_copy(items_hbm.at[wid], idx_v, sems[1])
        cp_u.wait()
        cp_v.wait()

        lane = lax.iota(jnp.int32, LANES)
        perms = [lane ^ d for d in (1, 2, 4, 8)]

        def block_base(i):
            return pl.multiple_of((i // 128) * 128, 128)

        def fire_batch(s, iv_u, iv_v, t0):
            for t in range(SPS):
                iu = iv_u[t0 + t]
                ivv = iv_v[t0 + t]
                pltpu.async_copy(
                    uft_hbm.at[:, pl.ds(block_base(iu), 128)],
                    u_bufs.at[s * SPS + t], sems[s])
                pltpu.async_copy(
                    ift_hbm.at[:, pl.ds(block_base(ivv), 128)],
                    v_bufs.at[s * SPS + t], sems[s])

        def drain_batch(s):
            for t in range(SPS):
                pltpu.make_async_copy(
                    uft_hbm.at[:, pl.ds(0, 128)], u_bufs.at[s * SPS + t],
                    sems[s]).wait()
                pltpu.make_async_copy(
                    ift_hbm.at[:, pl.ds(0, 128)], v_bufs.at[s * SPS + t],
                    sems[s]).wait()

        iv_u0 = idx_u[pl.ds(0, GRP)]
        iv_v0 = idx_v[pl.ds(0, GRP)]
        for s in range(NBUF):
            fire_batch(s, iv_u0, iv_v0, s * SPS)

        def grp(g, _):
            goff = g * GRP
            iv_u = idx_u[pl.ds(goff, GRP)]
            iv_v = idx_v[pl.ds(goff, GRP)]
            nof = jnp.minimum(g + 1, ngrp - 1) * GRP
            nu = idx_u[pl.ds(nof, GRP)]
            nv = idx_v[pl.ds(nof, GRP)]
            acc = jnp.zeros((LANES,), jnp.float32)
            for s in range(NBUF):
                drain_batch(s)
                for t in range(SPS):
                    j = s * SPS + t
                    iu = iv_u[j]
                    ivv = iv_v[j]
                    lu = jnp.broadcast_to(iu - block_base(iu), (LANES,))
                    lv = jnp.broadcast_to(ivv - block_base(ivv), (LANES,))
                    gu = plsc.load_gather(u_bufs.at[s * SPS + t], [lane, lu])
                    gv = plsc.load_gather(v_bufs.at[s * SPS + t], [lane, lv])
                    p = gu * gv
                    for perm in perms:
                        p = p + p.at[perm].get(mode="promise_in_bounds")
                    acc = jnp.where(lane == j, p, acc)

                @pl.when(g + 1 < ngrp)
                def _refire():
                    fire_batch(s, nu, nv, s * SPS)

            out_v[pl.ds(goff, GRP)] = acc
            return 0

        lax.fori_loop(0, ngrp, grp, 0)
        pltpu.sync_copy(out_v, out_hbm.at[pl.ds(base, bpw)])

    return body


@jax.jit
def kernel(x, user_factors, item_factors):
    nc, ns = _sc_dims()
    nw = nc * ns
    batch = x.shape[0]
    vocab = user_factors.shape[0]
    assert batch % (nw * GRP) == 0
    bpw = batch // nw

    users = x[:, 0].astype(jnp.int32).reshape(nw, bpw)
    items = x[:, 1].astype(jnp.int32).reshape(nw, bpw)

    mesh = plsc.VectorSubcoreMesh(core_axis_name="c", subcore_axis_name="s")
    fn = pl.kernel(
        _make_body(nc, bpw, vocab),
        out_type=jax.ShapeDtypeStruct((batch,), jnp.float32),
        mesh=mesh,
        scratch_types=[
            pltpu.VMEM((bpw,), jnp.int32),
            pltpu.VMEM((bpw,), jnp.int32),
            pltpu.VMEM((NBUF * SPS, NF, 128), jnp.float32),
            pltpu.VMEM((NBUF * SPS, NF, 128), jnp.float32),
            pltpu.VMEM((bpw,), jnp.float32),
        ] + [pltpu.SemaphoreType.DMA] * NBUF,
        compiler_params=pltpu.CompilerParams(
            disable_bounds_checks=True, needs_layout_passes=False),
    )
    return fn(users, items, user_factors.T, item_factors.T)
